# Initial kernel scaffold; baseline (speedup 1.0000x reference)
import jax
import jax.numpy as jnp
from jax.experimental import pallas as pl


def kernel(x, edge_attr, edge_index, emb_atom, emb_charge, emb_chiral, emb_aromatic, emb_ring, emb_bond_type, emb_bond_ring, W, b):
    raise NotImplementedError("write your pallas kernel here")


# speedup vs baseline: 3.9693x; 3.9693x over previous
import functools

import jax
import jax.numpy as jnp
from jax import lax
from jax.experimental import pallas as pl
from jax.experimental.pallas import tpu as pltpu
from jax.experimental.pallas import tpu_sc as plsc

N = 100000
OUT = 128
AC = 16

NC = 2
NS = 16
L = 16
NW = NC * NS

C = 160
K = 20
PER_W = C * K
NPAD = NW * PER_W


def _tables_body(ea, ec, ech, ear, er, w, b, t0, t12, t34):
    W = w[...]
    t0[...] = jnp.dot(ea[...], W[0:16, :],
                      preferred_element_type=jnp.float32) + b[...]
    p1 = jnp.dot(ec[...], W[16:32, :], preferred_element_type=jnp.float32)
    p2 = jnp.dot(ech[...], W[32:48, :], preferred_element_type=jnp.float32)
    t12[...] = p1[:, None, :] + p2[None, :, :]
    p3 = jnp.dot(ear[...], W[48:64, :], preferred_element_type=jnp.float32)
    p4 = jnp.dot(er[...], W[64:80, :], preferred_element_type=jnp.float32)
    t34[...] = p3[:, None, :] + p4[None, :, :]


_tc_tables = pl.pallas_call(
    _tables_body,
    out_shape=[
        jax.ShapeDtypeStruct((100, OUT), jnp.float32),
        jax.ShapeDtypeStruct((10, 10, OUT), jnp.float32),
        jax.ShapeDtypeStruct((10, 10, OUT), jnp.float32),
    ],
)


_mesh = plsc.VectorSubcoreMesh(core_axis_name="c", subcore_axis_name="s")


@functools.partial(
    pl.kernel,
    out_type=jax.ShapeDtypeStruct((NPAD * OUT,), jnp.float32),
    mesh=_mesh,
    compiler_params=pltpu.CompilerParams(needs_layout_passes=False),
    scratch_types=[
        pltpu.VMEM((C,), jnp.float32),
        pltpu.VMEM((C,), jnp.float32),
        pltpu.VMEM((C,), jnp.float32),
        pltpu.VMEM((C,), jnp.float32),
        pltpu.VMEM((C,), jnp.float32),
        pltpu.VMEM((C,), jnp.float32),
        pltpu.VMEM((100 * OUT,), jnp.float32),
        pltpu.VMEM((100 * OUT,), jnp.float32),
        pltpu.VMEM((100 * OUT,), jnp.float32),
        pltpu.VMEM((OUT,), jnp.float32),
        pltpu.VMEM((C * OUT,), jnp.float32),
    ],
)
def _sc_gather(x0_hbm, x1_hbm, x2_hbm, x3_hbm, x4_hbm, xc_hbm,
               t0_hbm, t12_hbm, t34_hbm, wl_hbm, out_hbm,
               x0v, x1v, x2v, x3v, x4v, xcv,
               t0v, t12v, t34v, wlv, obuf):
    wid = lax.axis_index("s") * NC + lax.axis_index("c")
    base = wid * PER_W
    pltpu.sync_copy(t0_hbm, t0v)
    pltpu.sync_copy(t12_hbm, t12v)
    pltpu.sync_copy(t34_hbm, t34v)
    pltpu.sync_copy(wl_hbm, wlv)
    iota = lax.iota(jnp.int32, L)
    wvecs = tuple(wlv[pl.ds(L * j, L)] for j in range(OUT // L))

    def chunk_body(ci, wv):
        off = base + ci * C
        pltpu.sync_copy(x0_hbm.at[pl.ds(off, C)], x0v)
        pltpu.sync_copy(x1_hbm.at[pl.ds(off, C)], x1v)
        pltpu.sync_copy(x2_hbm.at[pl.ds(off, C)], x2v)
        pltpu.sync_copy(x3_hbm.at[pl.ds(off, C)], x3v)
        pltpu.sync_copy(x4_hbm.at[pl.ds(off, C)], x4v)
        pltpu.sync_copy(xc_hbm.at[pl.ds(off, C)], xcv)

        def group_body(g, wv):
            s = pl.ds(g * L, L)
            c0 = x0v[s].astype(jnp.int32) * OUT
            c12 = (x1v[s].astype(jnp.int32) * 10 + x2v[s].astype(jnp.int32)) * OUT
            c34 = (x3v[s].astype(jnp.int32) * 10 + x4v[s].astype(jnp.int32)) * OUT
            xf = xcv[s]
            gbase = g * (L * OUT)
            for m in range(L):
                b0 = jnp.full((L,), c0[m], jnp.int32) + iota
                b12 = jnp.full((L,), c12[m], jnp.int32) + iota
                b34 = jnp.full((L,), c34[m], jnp.int32) + iota
                xn = jnp.full((L,), xf[m], jnp.float32)
                for j in range(OUT // L):
                    acc = plsc.load_gather(t0v, [b0 + (L * j)])
                    acc = acc + plsc.load_gather(t12v, [b12 + (L * j)])
                    acc = acc + plsc.load_gather(t34v, [b34 + (L * j)])
                    acc = acc + xn * wv[j]
                    obuf[pl.ds(gbase + m * OUT + L * j, L)] = acc
            return wv

        wv = lax.fori_loop(0, C // L, group_body, wv)
        pltpu.sync_copy(obuf, out_hbm.at[pl.ds(off * OUT, C * OUT)])
        return wv

    lax.fori_loop(0, K, chunk_body, wvecs)


def kernel(x, edge_attr, edge_index, emb_atom, emb_charge, emb_chiral,
           emb_aromatic, emb_ring, emb_bond_type, emb_bond_ring, W, b):
    t0, t12, t34 = _tc_tables(emb_atom, emb_charge, emb_chiral, emb_aromatic,
                              emb_ring, W, b.reshape(1, OUT))
    pad = NPAD - N
    cols = [jnp.pad(x[:, i], (0, pad)) for i in range(6)]
    outflat = _sc_gather(cols[0], cols[1], cols[2], cols[3], cols[4], cols[5],
                         t0.reshape(-1), t12.reshape(-1), t34.reshape(-1),
                         W[80])
    return outflat.reshape(NPAD, OUT)[:N]


# speedup vs baseline: 5.4890x; 1.3829x over previous
import functools

import jax
import jax.numpy as jnp
from jax import lax
from jax.experimental import pallas as pl
from jax.experimental.pallas import tpu as pltpu
from jax.experimental.pallas import tpu_sc as plsc

N = 100000
OUT = 128
AC = 16

NC = 2
NS = 16
L = 16
NW = NC * NS

C = 160
NCHUNK = N // C
KMAX = 20
W_FULL = NCHUNK - NW * (KMAX - 1)
XCH = 640
CW = 6 * C
CO = C * OUT


def _tables_body(ea, ec, ech, ear, er, w, b, t0, t12, t34):
    W = w[...]
    t0[...] = jnp.dot(ea[...], W[0:16, :],
                      preferred_element_type=jnp.float32) + b[...]
    p1 = jnp.dot(ec[...], W[16:32, :], preferred_element_type=jnp.float32)
    p2 = jnp.dot(ech[...], W[32:48, :], preferred_element_type=jnp.float32)
    t12[...] = p1[:, None, :] + p2[None, :, :]
    p3 = jnp.dot(ear[...], W[48:64, :], preferred_element_type=jnp.float32)
    p4 = jnp.dot(er[...], W[64:80, :], preferred_element_type=jnp.float32)
    t34[...] = p3[:, None, :] + p4[None, :, :]


_tc_tables = pl.pallas_call(
    _tables_body,
    out_shape=[
        jax.ShapeDtypeStruct((100, OUT), jnp.float32),
        jax.ShapeDtypeStruct((10, 10, OUT), jnp.float32),
        jax.ShapeDtypeStruct((10, 10, OUT), jnp.float32),
    ],
)


_mesh = plsc.VectorSubcoreMesh(core_axis_name="c", subcore_axis_name="s")


@functools.partial(
    pl.kernel,
    out_type=jax.ShapeDtypeStruct((N * OUT,), jnp.float32),
    mesh=_mesh,
    compiler_params=pltpu.CompilerParams(needs_layout_passes=False),
    scratch_types=[
        pltpu.VMEM((KMAX * CW,), jnp.float32),
        pltpu.VMEM((100 * OUT,), jnp.float32),
        pltpu.VMEM((100 * OUT,), jnp.float32),
        pltpu.VMEM((100 * OUT,), jnp.float32),
        pltpu.VMEM((OUT,), jnp.float32),
        pltpu.VMEM((CO,), jnp.float32),
        pltpu.VMEM((CO,), jnp.float32),
        pltpu.SemaphoreType.DMA,
        pltpu.SemaphoreType.DMA,
    ],
)
def _sc_gather(xp_hbm, t0_hbm, t12_hbm, t34_hbm, wl_hbm, out_hbm,
               xin, t0v, t12v, t34v, wlv, ob0, ob1, sem0, sem1):
    wid = lax.axis_index("s") * NC + lax.axis_index("c")
    kw = jnp.where(wid < W_FULL, KMAX, KMAX - 1)
    cbase = wid * KMAX - jnp.maximum(wid - W_FULL, 0)
    pltpu.sync_copy(t0_hbm, t0v)
    pltpu.sync_copy(t12_hbm, t12v)
    pltpu.sync_copy(t34_hbm, t34v)
    pltpu.sync_copy(wl_hbm, wlv)
    pltpu.sync_copy(xp_hbm.at[pl.ds(cbase * CW, KMAX * CW)], xin)
    iota = lax.iota(jnp.int32, L)
    wvecs = tuple(wlv[pl.ds(L * j, L)] for j in range(OUT // L))

    def compute_chunk(c, ob, wv):
        xoff = c * CW

        def group_body(g, wv):
            c0 = xin[pl.ds(xoff + g * L, L)].astype(jnp.int32) * OUT
            c12 = (xin[pl.ds(xoff + C + g * L, L)].astype(jnp.int32) * 10
                   + xin[pl.ds(xoff + 2 * C + g * L, L)].astype(jnp.int32)) * OUT
            c34 = (xin[pl.ds(xoff + 3 * C + g * L, L)].astype(jnp.int32) * 10
                   + xin[pl.ds(xoff + 4 * C + g * L, L)].astype(jnp.int32)) * OUT
            xf = xin[pl.ds(xoff + 5 * C + g * L, L)]
            gbase = g * (L * OUT)
            for m in range(L):
                b0 = jnp.full((L,), c0[m], jnp.int32) + iota
                b12 = jnp.full((L,), c12[m], jnp.int32) + iota
                b34 = jnp.full((L,), c34[m], jnp.int32) + iota
                xn = jnp.full((L,), xf[m], jnp.float32)
                for j in range(OUT // L):
                    acc = plsc.load_gather(t0v, [b0 + (L * j)])
                    acc = acc + plsc.load_gather(t12v, [b12 + (L * j)])
                    acc = acc + plsc.load_gather(t34v, [b34 + (L * j)])
                    acc = acc + xn * wv[j]
                    ob[pl.ds(gbase + m * OUT + L * j, L)] = acc
            return wv

        return lax.fori_loop(0, C // L, group_body, wv)

    def outer(i, wv):
        for b, (ob, sem) in enumerate(((ob0, sem0), (ob1, sem1))):
            c = 2 * i + b

            @pl.when(jnp.logical_and(c >= 2, c - 2 < kw))
            def _wait():
                pltpu.make_async_copy(ob, out_hbm.at[pl.ds(0, CO)], sem).wait()

            wv = compute_chunk(c, ob, wv)

            @pl.when(c < kw)
            def _start():
                pltpu.make_async_copy(
                    ob, out_hbm.at[pl.ds((cbase + c) * CO, CO)], sem).start()
        return wv

    lax.fori_loop(0, KMAX // 2, outer, wvecs)

    pltpu.make_async_copy(ob0, out_hbm.at[pl.ds(0, CO)], sem0).wait()

    @pl.when(kw == KMAX)
    def _tail():
        pltpu.make_async_copy(ob1, out_hbm.at[pl.ds(0, CO)], sem1).wait()


def kernel(x, edge_attr, edge_index, emb_atom, emb_charge, emb_chiral,
           emb_aromatic, emb_ring, emb_bond_type, emb_bond_ring, W, b):
    t0, t12, t34 = _tc_tables(emb_atom, emb_charge, emb_chiral, emb_aromatic,
                              emb_ring, W, b.reshape(1, OUT))
    xp = x.reshape(NCHUNK, C, 6).transpose(0, 2, 1).reshape(-1)
    xp = jnp.pad(xp, (0, (XCH - NCHUNK) * CW))
    outflat = _sc_gather(xp, t0.reshape(-1), t12.reshape(-1),
                         t34.reshape(-1), W[80])
    return outflat.reshape(N, OUT)
